# baseline (speedup 1.0000x reference)
import jax
import jax.numpy as jnp
from jax import lax
from jax.experimental import pallas as pl
from jax.experimental.pallas import tpu as pltpu
from jax.experimental.pallas import tpu_sc as plsc

N = 4096
D = 1024
F = 2048
E = 8
K = 2
TM = 512
NT = (N * K + E * TM) // TM
PR = NT * TM
TF = 512
CH = 128
SW = 256
NSUB = D // SW
WI = 128
AUX_COEF = 0.01
Z_COEF = 0.001


def _logits_body(x_ref, gw_ref, lg_ref):
    lg_ref[...] = lax.dot_general(x_ref[...], gw_ref[...],
                                  (((1,), (1,)), ((), ())),
                                  preferred_element_type=jnp.float32)


def _logits(xf, gate_w):
    return pl.pallas_call(
        _logits_body,
        grid=(N // TF,),
        in_specs=[
            pl.BlockSpec((TF, D), lambda t: (t, 0)),
            pl.BlockSpec((E, D), lambda t: (0, 0)),
        ],
        out_specs=pl.BlockSpec((TF, E), lambda t: (t, 0)),
        out_shape=jax.ShapeDtypeStruct((N, E), jnp.float32),
    )(xf, gate_w)


def _router_body(lg_ref, didx_ref, gidx_ref, wts_ref, meta_ref,
                 loss_ref):
    logits = lg_ref[...]
    zmax = jnp.max(logits, axis=1, keepdims=True)
    ex = jnp.exp(logits - zmax)
    sume = jnp.sum(ex, axis=1, keepdims=True)
    probs = ex / sume
    lse = jnp.log(sume) + zmax
    z_loss = Z_COEF * jnp.mean(lse * lse)
    pm = jnp.mean(probs, axis=0, keepdims=True)
    aux_loss = AUX_COEF * jnp.sum(pm * pm)
    loss_ref[...] = (aux_loss + z_loss).reshape(1, 1)

    eiota = lax.broadcasted_iota(jnp.int32, (N, E), 1)
    m1 = jnp.max(probs, axis=1, keepdims=True)
    i1 = jnp.min(jnp.where(probs == m1, eiota, E), axis=1, keepdims=True)
    probs2 = jnp.where(eiota == i1, -1.0, probs)
    m2 = jnp.max(probs2, axis=1, keepdims=True)
    i2 = jnp.min(jnp.where(probs2 == m2, eiota, E), axis=1, keepdims=True)
    s = m1 + m2
    wts_ref[...] = jnp.concatenate([m1 / s, m2 / s], axis=1)

    e_cm = jnp.concatenate([i1, i2], axis=0)
    oh = (e_cm == lax.broadcasted_iota(jnp.int32, (N * K, E), 1))
    oh = oh.astype(jnp.float32)

    r = lax.broadcasted_iota(jnp.int32, (CH, CH), 0)
    c = lax.broadcasted_iota(jnp.int32, (CH, CH), 1)
    tl = (c <= r).astype(jnp.float32)
    rank_rows = []
    run = jnp.zeros((1, E), jnp.float32)
    for ci in range((N * K) // CH):
        blk = oh[ci * CH:(ci + 1) * CH]
        s1 = lax.dot_general(tl, blk, (((1,), (0,)), ((), ())),
                             preferred_element_type=jnp.float32)
        rank_rows.append(s1 + run)
        run = run + jnp.sum(blk, axis=0, keepdims=True)
    rank = jnp.concatenate(rank_rows, axis=0)

    counts = jnp.sum(oh, axis=0, keepdims=True)
    lane8 = lax.broadcasted_iota(jnp.int32, (1, E), 1)
    tio = lax.broadcasted_iota(jnp.int32, (1, 64), 1)
    offs_row = jnp.zeros((1, E), jnp.float32)
    te_acc = jnp.zeros((1, 64), jnp.int32)
    end_acc = jnp.int32(0)
    for e in range(E):
        ce = jnp.sum(jnp.where(lane8 == e, counts, 0.0)).astype(jnp.int32)
        pe = ((ce + TM - 1) // TM) * TM
        offs_row = offs_row + jnp.where(
            lane8 == e, end_acc.astype(jnp.float32), 0.0)
        end_acc = end_acc + pe
        te_acc = te_acc + (tio * TM >= end_acc).astype(jnp.int32)
    nt = end_acc // TM
    te = jnp.minimum(te_acc, E - 1)
    meta_ref[...] = jnp.where(tio == NT, nt, te)

    off_j = jnp.sum(oh * offs_row, axis=1, keepdims=True)
    rank_j = jnp.sum(oh * rank, axis=1, keepdims=True)
    slot = (off_j + rank_j - 1.0).astype(jnp.int32)
    sub = PR * lax.broadcasted_iota(jnp.int32, (N, NSUB), 1)
    didx_ref[...] = jnp.concatenate(
        [slot[0:N] + sub, slot[N:2 * N] + sub], axis=1)
    gidx_ref[...] = jnp.concatenate(
        [slot + c * PR for c in range(NSUB)], axis=0)


def _router(lg):
    return pl.pallas_call(
        _router_body,
        out_shape=[
            jax.ShapeDtypeStruct((N, K * NSUB), jnp.int32),
            jax.ShapeDtypeStruct((NSUB * K * N, 1), jnp.int32),
            jax.ShapeDtypeStruct((N, K), jnp.float32),
            jax.ShapeDtypeStruct((1, 64), jnp.int32),
            jax.ShapeDtypeStruct((1, 1), jnp.float32),
        ],
    )(lg)


def _dispatch(x4, didx):
    mesh = plsc.VectorSubcoreMesh(core_axis_name="core",
                                  subcore_axis_name="subcore")

    @pl.kernel(out_type=jax.ShapeDtypeStruct((NSUB * PR, SW), jnp.float32),
               mesh=mesh)
    def dispatch_kernel(x_hbm, i_hbm, xs_hbm):
        def body(x_vmem, i_vmem):
            pltpu.sync_copy(x_vmem, xs_hbm.at[i_vmem.at[0]])
            pltpu.sync_copy(x_vmem, xs_hbm.at[i_vmem.at[1]])

        pltpu.emit_pipeline(
            body,
            grid=((N * NSUB) // WI,),
            in_specs=[
                pl.BlockSpec((WI, SW), lambda i: (i, 0)),
                pl.BlockSpec((K, WI), lambda i: (0, i)),
            ],
            out_specs=[],
            core_axis_name=("core", "subcore"),
            dimension_semantics=(pltpu.PARALLEL,),
        )(x_hbm, i_hbm)

    return dispatch_kernel(x4, didx)


def _ffn_body(te_ref, nt_ref, xs_ref, w1_ref, w2_ref, ys_ref):
    t = pl.program_id(0)

    @pl.when(t < nt_ref[0])
    def _():
        w1f = w1_ref[0]
        h = lax.dot_general(xs_ref[0], w1f[:, 0:SW],
                            (((1,), (1,)), ((), ())),
                            preferred_element_type=jnp.float32)
        for c in range(1, NSUB):
            h = h + lax.dot_general(xs_ref[c], w1f[:, c * SW:(c + 1) * SW],
                                    (((1,), (1,)), ((), ())),
                                    preferred_element_type=jnp.float32)
        h = h * jax.nn.sigmoid(h)
        y2 = lax.dot_general(h, w2_ref[0], (((1,), (1,)), ((), ())),
                             preferred_element_type=jnp.float32)
        for c in range(NSUB):
            ys_ref[c] = y2[:, c * SW:(c + 1) * SW]


def _ffn(te, nt, xs_cm, w1, w2):
    grid_spec = pltpu.PrefetchScalarGridSpec(
        num_scalar_prefetch=2,
        grid=(NT,),
        in_specs=[
            pl.BlockSpec((NSUB, TM, SW), lambda t, te, nt: (0, t, 0)),
            pl.BlockSpec((1, F, D), lambda t, te, nt: (te[t], 0, 0)),
            pl.BlockSpec((1, D, F), lambda t, te, nt: (te[t], 0, 0)),
        ],
        out_specs=pl.BlockSpec((NSUB, TM, SW), lambda t, te, nt: (0, t, 0)),
    )
    return pl.pallas_call(
        _ffn_body,
        grid_spec=grid_spec,
        out_shape=jax.ShapeDtypeStruct((NSUB, PR, SW), jnp.float32),
    )(te, nt, xs_cm, w1, w2)


def _gather(ys4, idx):
    mesh = plsc.VectorSubcoreMesh(core_axis_name="core",
                                  subcore_axis_name="subcore")

    @pl.kernel(out_type=jax.ShapeDtypeStruct((NSUB * K * N, SW),
                                             jnp.float32),
               mesh=mesh)
    def gather_kernel(ys_hbm, i_hbm, o_hbm):
        def body(i_vmem, o_vmem):
            pltpu.sync_copy(ys_hbm.at[i_vmem.at[0]], o_vmem)

        pltpu.emit_pipeline(
            body,
            grid=((K * N * NSUB) // WI,),
            in_specs=[pl.BlockSpec((1, WI), lambda i: (0, i))],
            out_specs=[pl.BlockSpec((WI, SW), lambda i: (i, 0))],
            core_axis_name=("core", "subcore"),
            dimension_semantics=(pltpu.PARALLEL,),
        )(i_hbm, o_hbm)

    return gather_kernel(ys4, idx)


def _shared_body(x_ref, sw1_ref, sw2_ref, sh_ref):
    xt = x_ref[...]
    h = lax.dot_general(xt, sw1_ref[0], (((1,), (1,)), ((), ())),
                        preferred_element_type=jnp.float32)
    h = h * jax.nn.sigmoid(h)
    sh_ref[...] = lax.dot_general(h, sw2_ref[0], (((1,), (1,)), ((), ())),
                                  preferred_element_type=jnp.float32)


def _shared(xf, sw1, sw2):
    return pl.pallas_call(
        _shared_body,
        grid=(N // TF,),
        in_specs=[
            pl.BlockSpec((TF, D), lambda t: (t, 0)),
            pl.BlockSpec((1, F, D), lambda t: (0, 0, 0)),
            pl.BlockSpec((1, D, F), lambda t: (0, 0, 0)),
        ],
        out_specs=pl.BlockSpec((TF, D), lambda t: (t, 0)),
        out_shape=jax.ShapeDtypeStruct((N, D), jnp.float32),
    )(xf, sw1, sw2)


def _finish_body(x_ref, yg_ref, wts_ref, sh_ref, a_ref, b_ref, y_ref):
    w = wts_ref[...]
    w0 = w[:, 0:1]
    w1c = w[:, 1:2]
    for c in range(NSUB):
        sl = slice(c * SW, (c + 1) * SW)
        routed = yg_ref[c, 0] * w0 + yg_ref[c, 1] * w1c
        y_ref[:, sl] = (a_ref[:, sl] * (sh_ref[:, sl] + routed)
                        + b_ref[:, sl] * x_ref[:, sl])


def _finish(xf, yg4, wts, sh, alpha, beta):
    return pl.pallas_call(
        _finish_body,
        grid=(N // TF,),
        in_specs=[
            pl.BlockSpec((TF, D), lambda t: (t, 0)),
            pl.BlockSpec((NSUB, K, TF, SW), lambda t: (0, 0, t, 0)),
            pl.BlockSpec((TF, K), lambda t: (t, 0)),
            pl.BlockSpec((TF, D), lambda t: (t, 0)),
            pl.BlockSpec((1, D), lambda t: (0, 0)),
            pl.BlockSpec((1, D), lambda t: (0, 0)),
        ],
        out_specs=pl.BlockSpec((TF, D), lambda t: (t, 0)),
        out_shape=jax.ShapeDtypeStruct((N, D), jnp.float32),
    )(xf, yg4, wts, sh, alpha, beta)


def kernel(x, gate_w, w1, w2, sw1, sw2, alpha, beta):
    Bs, Ts, Dd = x.shape
    xf = x.reshape(-1, Dd)
    didx, gidx, wts, meta, loss = _router(_logits(xf, gate_w))
    didx_kn = didx.reshape(N, K, NSUB).transpose(1, 0, 2).reshape(K, N * NSUB)
    xs4 = _dispatch(xf.reshape(N * NSUB, SW), didx_kn)
    sh = _shared(xf, sw1, sw2)
    te = meta[0, :NT]
    nt = meta[0, NT:NT + 1]
    ys = _ffn(te, nt, xs4.reshape(NSUB, PR, SW), w1, w2)
    yg = _gather(ys.reshape(NSUB * PR, SW), gidx.reshape(1, NSUB * K * N))
    y = _finish(xf, yg.reshape(NSUB, K, N, SW), wts, sh,
                alpha.reshape(1, Dd), beta.reshape(1, Dd))
    return y.reshape(Bs, Ts, Dd), loss[0, 0]


# speedup vs baseline: 1.0551x; 1.0551x over previous
import jax
import jax.numpy as jnp
from jax import lax
from jax.experimental import pallas as pl
from jax.experimental.pallas import tpu as pltpu
from jax.experimental.pallas import tpu_sc as plsc

N = 4096
D = 1024
F = 2048
E = 8
K = 2
TM = 512
NT = (N * K + E * TM) // TM
PR = NT * TM
TF = 512
CH = 128
SW = 256
NSUB = D // SW
WI = 128
AUX_COEF = 0.01
Z_COEF = 0.001


def _router_body(x_ref, gw_ref, didx_ref, wts_ref, meta_ref, loss_ref):
    x = x_ref[...]
    gw = gw_ref[...]
    logits = lax.dot_general(x, gw, (((1,), (1,)), ((), ())),
                             preferred_element_type=jnp.float32)
    zmax = jnp.max(logits, axis=1, keepdims=True)
    ex = jnp.exp(logits - zmax)
    sume = jnp.sum(ex, axis=1, keepdims=True)
    probs = ex / sume
    lse = jnp.log(sume) + zmax
    z_loss = Z_COEF * jnp.mean(lse * lse)
    pm = jnp.mean(probs, axis=0, keepdims=True)
    aux_loss = AUX_COEF * jnp.sum(pm * pm)
    loss_ref[...] = (aux_loss + z_loss).reshape(1, 1)

    eiota = lax.broadcasted_iota(jnp.int32, (N, E), 1)
    m1 = jnp.max(probs, axis=1, keepdims=True)
    i1 = jnp.min(jnp.where(probs == m1, eiota, E), axis=1, keepdims=True)
    probs2 = jnp.where(eiota == i1, -1.0, probs)
    m2 = jnp.max(probs2, axis=1, keepdims=True)
    i2 = jnp.min(jnp.where(probs2 == m2, eiota, E), axis=1, keepdims=True)
    s = m1 + m2
    wts_ref[...] = jnp.concatenate([m1 / s, m2 / s], axis=1)

    e_cm = jnp.concatenate([i1, i2], axis=0)
    oh = (e_cm == lax.broadcasted_iota(jnp.int32, (N * K, E), 1))
    oh = oh.astype(jnp.float32)

    r = lax.broadcasted_iota(jnp.int32, (CH, CH), 0)
    c = lax.broadcasted_iota(jnp.int32, (CH, CH), 1)
    tl = (c <= r).astype(jnp.float32)
    rank_rows = []
    run = jnp.zeros((1, E), jnp.float32)
    for ci in range((N * K) // CH):
        blk = oh[ci * CH:(ci + 1) * CH]
        s1 = lax.dot_general(tl, blk, (((1,), (0,)), ((), ())),
                             preferred_element_type=jnp.float32)
        rank_rows.append(s1 + run)
        run = run + jnp.sum(blk, axis=0, keepdims=True)
    rank = jnp.concatenate(rank_rows, axis=0)

    counts = jnp.sum(oh, axis=0, keepdims=True)
    lane8 = lax.broadcasted_iota(jnp.int32, (1, E), 1)
    tio = lax.broadcasted_iota(jnp.int32, (1, 64), 1)
    offs_row = jnp.zeros((1, E), jnp.float32)
    te_acc = jnp.zeros((1, 64), jnp.int32)
    end_acc = jnp.int32(0)
    for e in range(E):
        ce = jnp.sum(jnp.where(lane8 == e, counts, 0.0)).astype(jnp.int32)
        pe = ((ce + TM - 1) // TM) * TM
        offs_row = offs_row + jnp.where(
            lane8 == e, end_acc.astype(jnp.float32), 0.0)
        end_acc = end_acc + pe
        te_acc = te_acc + (tio * TM >= end_acc).astype(jnp.int32)
    nt = end_acc // TM
    te = jnp.minimum(te_acc, E - 1)
    meta_ref[...] = jnp.where(tio == NT, nt, te)

    off_j = jnp.sum(oh * offs_row, axis=1, keepdims=True)
    rank_j = jnp.sum(oh * rank, axis=1, keepdims=True)
    slot = (off_j + rank_j - 1.0).astype(jnp.int32)
    sub = PR * lax.broadcasted_iota(jnp.int32, (N, NSUB), 1)
    didx_ref[...] = jnp.concatenate(
        [slot[0:N] + sub, slot[N:2 * N] + sub], axis=1)


def _router(xf, gate_w):
    return pl.pallas_call(
        _router_body,
        out_shape=[
            jax.ShapeDtypeStruct((N, K * NSUB), jnp.int32),
            jax.ShapeDtypeStruct((N, K), jnp.float32),
            jax.ShapeDtypeStruct((1, 64), jnp.int32),
            jax.ShapeDtypeStruct((1, 1), jnp.float32),
        ],
    )(xf, gate_w)


def _dispatch(x4, didx):
    mesh = plsc.VectorSubcoreMesh(core_axis_name="core",
                                  subcore_axis_name="subcore")

    @pl.kernel(out_type=jax.ShapeDtypeStruct((NSUB * PR, SW), jnp.float32),
               mesh=mesh)
    def dispatch_kernel(x_hbm, i_hbm, xs_hbm):
        def body(x_vmem, i_vmem):
            pltpu.sync_copy(x_vmem, xs_hbm.at[i_vmem.at[0]])
            pltpu.sync_copy(x_vmem, xs_hbm.at[i_vmem.at[1]])

        pltpu.emit_pipeline(
            body,
            grid=((N * NSUB) // WI,),
            in_specs=[
                pl.BlockSpec((WI, SW), lambda i: (i, 0)),
                pl.BlockSpec((K, WI), lambda i: (0, i)),
            ],
            out_specs=[],
            core_axis_name=("core", "subcore"),
            dimension_semantics=(pltpu.PARALLEL,),
        )(x_hbm, i_hbm)

    return dispatch_kernel(x4, didx)


def _ffn_body(te_ref, nt_ref, xs_ref, w1_ref, w2_ref, ys_ref):
    t = pl.program_id(0)

    @pl.when(t < nt_ref[0])
    def _():
        w1f = w1_ref[0]
        h = lax.dot_general(xs_ref[0], w1f[:, 0:SW],
                            (((1,), (1,)), ((), ())),
                            preferred_element_type=jnp.float32)
        for c in range(1, NSUB):
            h = h + lax.dot_general(xs_ref[c], w1f[:, c * SW:(c + 1) * SW],
                                    (((1,), (1,)), ((), ())),
                                    preferred_element_type=jnp.float32)
        h = h * jax.nn.sigmoid(h)
        y2 = lax.dot_general(h, w2_ref[0], (((1,), (1,)), ((), ())),
                             preferred_element_type=jnp.float32)
        for c in range(NSUB):
            ys_ref[c] = y2[:, c * SW:(c + 1) * SW]


def _ffn(te, nt, xs_cm, w1, w2):
    grid_spec = pltpu.PrefetchScalarGridSpec(
        num_scalar_prefetch=2,
        grid=(NT,),
        in_specs=[
            pl.BlockSpec((NSUB, TM, SW), lambda t, te, nt: (0, t, 0)),
            pl.BlockSpec((1, F, D), lambda t, te, nt: (te[t], 0, 0)),
            pl.BlockSpec((1, D, F), lambda t, te, nt: (te[t], 0, 0)),
        ],
        out_specs=pl.BlockSpec((NSUB, TM, SW), lambda t, te, nt: (0, t, 0)),
    )
    return pl.pallas_call(
        _ffn_body,
        grid_spec=grid_spec,
        out_shape=jax.ShapeDtypeStruct((NSUB, PR, SW), jnp.float32),
    )(te, nt, xs_cm, w1, w2)


def _gather(ys4, idx):
    mesh = plsc.VectorSubcoreMesh(core_axis_name="core",
                                  subcore_axis_name="subcore")

    @pl.kernel(out_type=jax.ShapeDtypeStruct((NSUB * K * N, SW),
                                             jnp.float32),
               mesh=mesh)
    def gather_kernel(ys_hbm, i_hbm, o_hbm):
        def body(i_vmem, o_vmem):
            pltpu.sync_copy(ys_hbm.at[i_vmem.at[0]], o_vmem)

        pltpu.emit_pipeline(
            body,
            grid=((K * N * NSUB) // WI,),
            in_specs=[pl.BlockSpec(
                (1, WI),
                lambda i: (((i // 32) % K) * NSUB + i // 64, i % 32))],
            out_specs=[pl.BlockSpec((WI, SW), lambda i: (i, 0))],
            core_axis_name=("core", "subcore"),
            dimension_semantics=(pltpu.PARALLEL,),
        )(i_hbm, o_hbm)

    return gather_kernel(ys4, idx)


def _shared_body(x_ref, sw1_ref, sw2_ref, sh_ref):
    xt = x_ref[...]
    h = lax.dot_general(xt, sw1_ref[0], (((1,), (1,)), ((), ())),
                        preferred_element_type=jnp.float32)
    h = h * jax.nn.sigmoid(h)
    sh_ref[...] = lax.dot_general(h, sw2_ref[0], (((1,), (1,)), ((), ())),
                                  preferred_element_type=jnp.float32)


def _shared(xf, sw1, sw2):
    return pl.pallas_call(
        _shared_body,
        grid=(N // TF,),
        in_specs=[
            pl.BlockSpec((TF, D), lambda t: (t, 0)),
            pl.BlockSpec((1, F, D), lambda t: (0, 0, 0)),
            pl.BlockSpec((1, D, F), lambda t: (0, 0, 0)),
        ],
        out_specs=pl.BlockSpec((TF, D), lambda t: (t, 0)),
        out_shape=jax.ShapeDtypeStruct((N, D), jnp.float32),
    )(xf, sw1, sw2)


def _finish_body(x_ref, yg_ref, wts_ref, sh_ref, a_ref, b_ref, y_ref):
    w = wts_ref[...]
    w0 = w[:, 0:1]
    w1c = w[:, 1:2]
    for c in range(NSUB):
        sl = slice(c * SW, (c + 1) * SW)
        routed = yg_ref[c, 0] * w0 + yg_ref[c, 1] * w1c
        y_ref[:, sl] = (a_ref[:, sl] * (sh_ref[:, sl] + routed)
                        + b_ref[:, sl] * x_ref[:, sl])


def _finish(xf, yg4, wts, sh, alpha, beta):
    return pl.pallas_call(
        _finish_body,
        grid=(N // TF,),
        in_specs=[
            pl.BlockSpec((TF, D), lambda t: (t, 0)),
            pl.BlockSpec((NSUB, K, TF, SW), lambda t: (0, 0, t, 0)),
            pl.BlockSpec((TF, K), lambda t: (t, 0)),
            pl.BlockSpec((TF, D), lambda t: (t, 0)),
            pl.BlockSpec((1, D), lambda t: (0, 0)),
            pl.BlockSpec((1, D), lambda t: (0, 0)),
        ],
        out_specs=pl.BlockSpec((TF, D), lambda t: (t, 0)),
        out_shape=jax.ShapeDtypeStruct((N, D), jnp.float32),
    )(xf, yg4, wts, sh, alpha, beta)


def kernel(x, gate_w, w1, w2, sw1, sw2, alpha, beta):
    Bs, Ts, Dd = x.shape
    xf = x.reshape(-1, Dd)
    didx, wts, meta, loss = _router(xf, gate_w)
    didx_kn = didx.reshape(N, K, NSUB).transpose(1, 0, 2).reshape(K, N * NSUB)
    xs4 = _dispatch(xf.reshape(N * NSUB, SW), didx_kn)
    sh = _shared(xf, sw1, sw2)
    te = meta[0, :NT]
    nt = meta[0, NT:NT + 1]
    ys = _ffn(te, nt, xs4.reshape(NSUB, PR, SW), w1, w2)
    yg = _gather(ys.reshape(NSUB * PR, SW), didx.transpose(1, 0))
    y = _finish(xf, yg.reshape(NSUB, K, N, SW), wts, sh,
                alpha.reshape(1, Dd), beta.reshape(1, Dd))
    return y.reshape(Bs, Ts, Dd), loss[0, 0]
